# baseline (speedup 1.0000x reference)
import jax
import jax.numpy as jnp
from jax import lax
from jax.experimental import pallas as pl
from jax.experimental.pallas import tpu as pltpu
from jax.experimental.pallas import tpu_sc as plsc

_B, _E, _H, _W = 32, 768, 16, 16
_HW = _H * _W
_NC, _NS = 2, 16
_NW = _NC * _NS
_EC = _E // _NW


def _sc_body(row_hbm, col_hbm, out_hbm, row_v, col_v, pos_v, sem):
    wid = lax.axis_index("s") * _NC + lax.axis_index("c")
    e0 = wid * _EC
    pltpu.sync_copy(row_hbm.at[pl.ds(0, _H * _E)], row_v)
    pltpu.sync_copy(col_hbm.at[pl.ds(0, _W * _E)], col_v)
    lanes = lax.broadcasted_iota(jnp.int32, (16,), 0)
    for i in range(_EC):
        e = e0 + i
        idx = lanes * _E + jnp.full((16,), e, dtype=jnp.int32)
        colvec = plsc.load_gather(col_v, [idx])
        rowvec = plsc.load_gather(row_v, [idx])
        for h in range(_H):
            pos_v[pl.ds(i * _HW + h * _W, _W)] = colvec + rowvec[h]
    for b in range(_B):
        pltpu.make_async_copy(
            pos_v, out_hbm.at[b, pl.ds(e0 * _HW, _EC * _HW)], sem
        ).start()
    for b in range(_B):
        pltpu.make_async_copy(
            pos_v, out_hbm.at[b, pl.ds(e0 * _HW, _EC * _HW)], sem
        ).wait()


def kernel(x, row_table, col_table):
    B, E, H, W = x.shape
    mesh = plsc.VectorSubcoreMesh(
        core_axis_name="c", subcore_axis_name="s",
        num_cores=_NC, num_subcores=_NS,
    )
    out2 = pl.kernel(
        _sc_body,
        out_type=jax.ShapeDtypeStruct((_B, _E * _HW), jnp.float32),
        mesh=mesh,
        compiler_params=pltpu.CompilerParams(
            use_tc_tiling_on_sc=False, needs_layout_passes=False
        ),
        scratch_types=[
            pltpu.VMEM((_H * _E,), jnp.float32),
            pltpu.VMEM((_W * _E,), jnp.float32),
            pltpu.VMEM((_EC * _HW,), jnp.float32),
            pltpu.SemaphoreType.DMA,
        ],
    )(row_table.reshape(-1), col_table.reshape(-1))
    return out2.reshape(B, E, H, W)


# speedup vs baseline: 2.5072x; 2.5072x over previous
import jax
import jax.numpy as jnp
from jax import lax
from jax.experimental import pallas as pl
from jax.experimental.pallas import tpu as pltpu
from jax.experimental.pallas import tpu_sc as plsc

_B, _E, _H, _W = 32, 768, 16, 16
_HW = _H * _W
_NC, _NS = 2, 16
_NW = _NC * _NS
_EC = _E // _NW


def _sc_body(row_hbm, col_hbm, out_hbm, row_v, col_v, pos_v, sem):
    wid = lax.axis_index("s") * _NC + lax.axis_index("c")
    e0 = wid * _EC
    pltpu.sync_copy(row_hbm.at[pl.ds(0, _H)], row_v)
    pltpu.sync_copy(col_hbm.at[pl.ds(0, _W)], col_v)
    lanes = lax.broadcasted_iota(jnp.int32, (16,), 0)
    for i in range(_EC):
        evec = jnp.full((16,), e0 + i, dtype=jnp.int32)
        colvec = plsc.load_gather(col_v, [lanes, evec])
        rowvec = plsc.load_gather(row_v, [lanes, evec])
        for h in range(_H):
            pos_v[i, pl.ds(h * _W, _W)] = colvec + rowvec[h]
    for b in range(_B):
        pltpu.make_async_copy(
            pos_v, out_hbm.at[b, pl.ds(e0, _EC)], sem
        ).start()
    for b in range(_B):
        pltpu.make_async_copy(
            pos_v, out_hbm.at[b, pl.ds(e0, _EC)], sem
        ).wait()


def kernel(x, row_table, col_table):
    B, E, H, W = x.shape
    mesh = plsc.VectorSubcoreMesh(
        core_axis_name="c", subcore_axis_name="s",
        num_cores=_NC, num_subcores=_NS,
    )
    out3 = pl.kernel(
        _sc_body,
        out_type=jax.ShapeDtypeStruct((_B, _E, _HW), jnp.float32),
        mesh=mesh,
        compiler_params=pltpu.CompilerParams(
            use_tc_tiling_on_sc=False, needs_layout_passes=False
        ),
        scratch_types=[
            pltpu.VMEM((_H, _E), jnp.float32),
            pltpu.VMEM((_W, _E), jnp.float32),
            pltpu.VMEM((_EC, _HW), jnp.float32),
            pltpu.SemaphoreType.DMA,
        ],
    )(row_table, col_table)
    return out3.reshape(B, E, H, W)


# speedup vs baseline: 6.0220x; 2.4019x over previous
import jax
import jax.numpy as jnp
from jax.experimental import pallas as pl
from jax.experimental.pallas import tpu as pltpu

_B = 32


_GRP = 4
_NSEM = 8


def _tc_body(row_ref, col_ref, o_hbm, pos_vmem, sems):
    row16 = row_ref[...]
    col16 = col_ref[...]
    hw = jax.lax.broadcasted_iota(jnp.int32, (16, 256), 1)
    lane = jax.lax.broadcasted_iota(jnp.int32, (16, 256), 0)
    sh = (lane == hw // 16).astype(jnp.float32)
    sw = (lane == hw % 16).astype(jnp.float32)
    dims = (((0,), (0,)), ((), ()))
    pos = jax.lax.dot_general(row16, sh, dims, preferred_element_type=jnp.float32)
    pos = pos + jax.lax.dot_general(col16, sw, dims, preferred_element_type=jnp.float32)
    ngrp = _B // _GRP
    for g in range(ngrp):
        for j in range(_GRP):
            pos_vmem[g * _GRP + j] = pos
        pltpu.make_async_copy(
            pos_vmem.at[pl.ds(g * _GRP, _GRP)],
            o_hbm.at[pl.ds(g * _GRP, _GRP)],
            sems.at[g % _NSEM],
        ).start()
    for g in range(ngrp):
        pltpu.make_async_copy(
            pos_vmem.at[pl.ds(g * _GRP, _GRP)],
            o_hbm.at[pl.ds(g * _GRP, _GRP)],
            sems.at[g % _NSEM],
        ).wait()


def kernel(x, row_table, col_table):
    B, E, H, W = x.shape
    out3 = pl.pallas_call(
        _tc_body,
        in_specs=[
            pl.BlockSpec((H, E), lambda: (0, 0)),
            pl.BlockSpec((W, E), lambda: (0, 0)),
        ],
        out_specs=pl.BlockSpec(memory_space=pltpu.MemorySpace.HBM),
        compiler_params=pltpu.CompilerParams(
            disable_bounds_checks=True, skip_device_barrier=True
        ),
        out_shape=jax.ShapeDtypeStruct((B, E, H * W), jnp.float32),
        scratch_shapes=[
            pltpu.VMEM((B, E, H * W), jnp.float32),
            pltpu.SemaphoreType.DMA((_NSEM,)),
        ],
    )(row_table[:H], col_table[:W])
    return out3.reshape(B, E, H, W)


# speedup vs baseline: 16.7495x; 2.7814x over previous
import jax
import jax.numpy as jnp
from jax.experimental import pallas as pl
from jax.experimental.pallas import tpu as pltpu

_B, _E, _H, _W = 32, 768, 16, 16
_HW = _H * _W
_GRP = 4
_NSEM = 8


def _tc_body(row_ref, col_ref, o_hbm, img, sems):
    row16 = row_ref[...]
    col16 = col_ref[...]
    hwi = jax.lax.broadcasted_iota(jnp.int32, (_HW, _H), 0)
    lane = jax.lax.broadcasted_iota(jnp.int32, (_HW, _H), 1)
    rh = (lane == hwi // _W).astype(jnp.float32)
    rw = (lane == hwi % _W).astype(jnp.float32)
    dims = (((1,), (0,)), ((), ()))
    pos2 = jax.lax.dot_general(rh, row16, dims, preferred_element_type=jnp.float32)
    pos2 = pos2 + jax.lax.dot_general(rw, col16, dims, preferred_element_type=jnp.float32)
    for g in range(_B // _GRP):
        for j in range(_GRP):
            img[g * _GRP + j] = pos2
        pltpu.make_async_copy(
            img.at[pl.ds(g * _GRP, _GRP)],
            o_hbm.at[pl.ds(g * _GRP, _GRP)],
            sems.at[g % _NSEM],
        ).start()
    for g in range(_B // _GRP):
        pltpu.make_async_copy(
            img.at[pl.ds(g * _GRP, _GRP)],
            o_hbm.at[pl.ds(g * _GRP, _GRP)],
            sems.at[g % _NSEM],
        ).wait()


def kernel(x, row_table, col_table):
    B, E, H, W = x.shape
    out3 = pl.pallas_call(
        _tc_body,
        in_specs=[
            pl.BlockSpec((H, E), lambda: (0, 0)),
            pl.BlockSpec((W, E), lambda: (0, 0)),
        ],
        out_specs=pl.BlockSpec(memory_space=pltpu.MemorySpace.HBM),
        out_shape=jax.ShapeDtypeStruct((B, H * W, E), jnp.float32),
        scratch_shapes=[
            pltpu.VMEM((B, H * W, E), jnp.float32),
            pltpu.SemaphoreType.DMA((_NSEM,)),
        ],
    )(row_table[:H], col_table[:W])
    return jnp.transpose(out3.reshape(B, H, W, E), (0, 3, 1, 2))


# speedup vs baseline: 16.9625x; 1.0127x over previous
import jax
import jax.numpy as jnp
from jax.experimental import pallas as pl
from jax.experimental.pallas import tpu as pltpu

_B, _E, _H, _W = 32, 768, 16, 16
_HW = _H * _W
_GRP = 4
_NSEM = 8


def _tc_body(row_ref, col_ref, o_hbm, img, sems):
    row16 = row_ref[...]
    col16 = col_ref[...]
    pos2 = jnp.repeat(row16, _W, axis=0) + jnp.tile(col16, (_H, 1))
    for g in range(_B // _GRP):
        for j in range(_GRP):
            img[g * _GRP + j] = pos2
        pltpu.make_async_copy(
            img.at[pl.ds(g * _GRP, _GRP)],
            o_hbm.at[pl.ds(g * _GRP, _GRP)],
            sems.at[g % _NSEM],
        ).start()
    for g in range(_B // _GRP):
        pltpu.make_async_copy(
            img.at[pl.ds(g * _GRP, _GRP)],
            o_hbm.at[pl.ds(g * _GRP, _GRP)],
            sems.at[g % _NSEM],
        ).wait()


def kernel(x, row_table, col_table):
    B, E, H, W = x.shape
    out3 = pl.pallas_call(
        _tc_body,
        in_specs=[
            pl.BlockSpec((H, E), lambda: (0, 0)),
            pl.BlockSpec((W, E), lambda: (0, 0)),
        ],
        out_specs=pl.BlockSpec(memory_space=pltpu.MemorySpace.HBM),
        out_shape=jax.ShapeDtypeStruct((B, H * W, E), jnp.float32),
        scratch_shapes=[
            pltpu.VMEM((B, H * W, E), jnp.float32),
            pltpu.SemaphoreType.DMA((_NSEM,)),
        ],
    )(row_table[:H], col_table[:W])
    return jnp.transpose(out3.reshape(B, H, W, E), (0, 3, 1, 2))


# speedup vs baseline: 22.4800x; 1.3253x over previous
import jax
import jax.numpy as jnp
from jax.experimental import pallas as pl
from jax.experimental.pallas import tpu as pltpu

_B, _E, _H, _W = 32, 768, 16, 16
_HW = _H * _W
_GRP = 4
_NSEM = 8


def _tc_body(row_ref, col_ref, o_hbm, img, sems):
    row16 = row_ref[...]
    col16 = col_ref[...]
    pos2 = jnp.repeat(row16, _W, axis=0) + jnp.tile(col16, (_H, 1))
    groups = [(0, 1), (1, _GRP - 1)] + [(b, _GRP) for b in range(_GRP, _B, _GRP)]
    for k, (b0, n) in enumerate(groups):
        for j in range(n):
            img[b0 + j] = pos2
        pltpu.make_async_copy(
            img.at[pl.ds(b0, n)], o_hbm.at[pl.ds(b0, n)], sems.at[k % _NSEM]
        ).start()
    for k, (b0, n) in enumerate(groups):
        pltpu.make_async_copy(
            img.at[pl.ds(b0, n)], o_hbm.at[pl.ds(b0, n)], sems.at[k % _NSEM]
        ).wait()


def kernel(x, row_table, col_table):
    B, E, H, W = x.shape
    out3 = pl.pallas_call(
        _tc_body,
        grid=(1,),
        in_specs=[
            pl.BlockSpec((H, E), lambda i: (0, 0)),
            pl.BlockSpec((W, E), lambda i: (0, 0)),
        ],
        out_specs=pl.BlockSpec(memory_space=pltpu.MemorySpace.HBM),
        out_shape=jax.ShapeDtypeStruct((B, H * W, E), jnp.float32),
        scratch_shapes=[
            pltpu.VMEM((B, H * W, E), jnp.float32),
            pltpu.SemaphoreType.DMA((_NSEM,)),
        ],
    )(row_table, col_table)
    return jnp.transpose(out3.reshape(B, H, W, E), (0, 3, 1, 2))
